# baseline (speedup 1.0000x reference)
import functools

import jax
import jax.numpy as jnp
from jax import lax
from jax.experimental import pallas as pl
from jax.experimental.pallas import tpu as pltpu
from jax.experimental.pallas import tpu_sc as plsc

_ROWS = 16384
_COLS = 200
_N = _ROWS * _COLS
_NC = 2
_NS = 16
_NW = _NC * _NS
_PER_W = _N // _NW
_CHUNK = 25600
_NCHUNK = _PER_W // _CHUNK
_LANES = 16

_mesh = plsc.VectorSubcoreMesh(core_axis_name="c", subcore_axis_name="s")


@functools.partial(
    pl.kernel,
    mesh=_mesh,
    out_type=jax.ShapeDtypeStruct((_N,), jnp.float32),
    scratch_types=[
        pltpu.VMEM((_LANES,), jnp.float32),
        pltpu.VMEM((_CHUNK,), jnp.int32),
        pltpu.VMEM((_CHUNK,), jnp.float32),
    ],
)
def _lookup(keys_hbm, vals_hbm, out_hbm, vals_v, in_v, out_v):
    wid = lax.axis_index("s") * _NC + lax.axis_index("c")
    base = wid * _PER_W
    pltpu.sync_copy(vals_hbm, vals_v)
    vals_vec = vals_v[...]

    def chunk_body(ci, carry):
        off = base + ci * _CHUNK
        pltpu.sync_copy(keys_hbm.at[pl.ds(off, _CHUNK)], in_v)

        def vec_body(i, carry2):
            sl = pl.ds(i * _LANES, _LANES)
            x = in_v[sl]
            u = plsc.bitcast(x, jnp.uint32)
            idx = plsc.bitcast(jnp.minimum(u, jnp.uint32(3)), jnp.int32)
            out_v[sl] = lax.gather(
                vals_vec,
                idx[:, None],
                dimension_numbers=lax.GatherDimensionNumbers(
                    offset_dims=(),
                    collapsed_slice_dims=(0,),
                    start_index_map=(0,),
                ),
                slice_sizes=(1,),
                mode=lax.GatherScatterMode.PROMISE_IN_BOUNDS,
            )
            return carry2

        lax.fori_loop(0, _CHUNK // _LANES, vec_body, 0, unroll=8)
        pltpu.sync_copy(out_v, out_hbm.at[pl.ds(off, _CHUNK)])
        return carry

    lax.fori_loop(0, _NCHUNK, chunk_body, 0)


def kernel(inputs, values):
    flat = inputs.reshape(-1).astype(jnp.int32)
    vals16 = jnp.zeros((_LANES,), jnp.float32).at[:4].set(values)
    out = _lookup(flat, vals16)
    return out.reshape(inputs.shape)


# speedup vs baseline: 1.2851x; 1.2851x over previous
import functools

import jax
import jax.numpy as jnp
from jax import lax
from jax.experimental import pallas as pl
from jax.experimental.pallas import tpu as pltpu
from jax.experimental.pallas import tpu_sc as plsc

_ROWS = 16384
_COLS = 200
_N = _ROWS * _COLS
_NC = 2
_NS = 16
_NW = _NC * _NS
_PER_W = _N // _NW
_CHUNK = 12800
_NCHUNK = _PER_W // _CHUNK
_LANES = 16
_GDN = lax.GatherDimensionNumbers(
    offset_dims=(), collapsed_slice_dims=(0,), start_index_map=(0,))

_mesh = plsc.VectorSubcoreMesh(core_axis_name="c", subcore_axis_name="s")


@functools.partial(
    pl.kernel,
    mesh=_mesh,
    out_type=jax.ShapeDtypeStruct((_N,), jnp.float32),
    scratch_types=[
        pltpu.VMEM((_LANES,), jnp.float32),
        pltpu.VMEM((2, _CHUNK), jnp.int32),
        pltpu.VMEM((2, _CHUNK), jnp.float32),
        pltpu.SemaphoreType.DMA,
        pltpu.SemaphoreType.DMA,
        pltpu.SemaphoreType.DMA,
        pltpu.SemaphoreType.DMA,
    ],
)
def _lookup(keys_hbm, vals_hbm, out_hbm, vals_v, in_v, out_v,
            isem0, isem1, osem0, osem1):
    wid = lax.axis_index("s") * _NC + lax.axis_index("c")
    base = wid * _PER_W
    isems = (isem0, isem1)
    osems = (osem0, osem1)

    pltpu.sync_copy(vals_hbm, vals_v)
    vals_vec = vals_v[...]

    def in_copy(ci):
        b = ci % 2
        return pltpu.make_async_copy(
            keys_hbm.at[pl.ds(base + ci * _CHUNK, _CHUNK)],
            in_v.at[b], isems[b])

    def out_copy(ci):
        b = ci % 2
        return pltpu.make_async_copy(
            out_v.at[b], out_hbm.at[pl.ds(base + ci * _CHUNK, _CHUNK)],
            osems[b])

    in_copy(0).start()
    for ci in range(_NCHUNK):
        b = ci % 2
        if ci + 1 < _NCHUNK:
            in_copy(ci + 1).start()
        in_copy(ci).wait()
        if ci >= 2:
            out_copy(ci - 2).wait()
        in_ref = in_v.at[b]
        out_ref = out_v.at[b]

        @plsc.parallel_loop(0, _CHUNK, step=_LANES, unroll=8)
        def vec_body(i):
            sl = pl.ds(i, _LANES)
            x = in_ref[sl]
            u = plsc.bitcast(x, jnp.uint32)
            idx = plsc.bitcast(jnp.minimum(u, jnp.uint32(3)), jnp.int32)
            out_ref[sl] = lax.gather(
                vals_vec, idx[:, None], dimension_numbers=_GDN,
                slice_sizes=(1,),
                mode=lax.GatherScatterMode.PROMISE_IN_BOUNDS)

        out_copy(ci).start()
    out_copy(_NCHUNK - 2).wait()
    out_copy(_NCHUNK - 1).wait()


def kernel(inputs, values):
    flat = inputs.reshape(-1).astype(jnp.int32)
    vals16 = jnp.zeros((_LANES,), jnp.float32).at[:4].set(values)
    out = _lookup(flat, vals16)
    return out.reshape(inputs.shape)


# speedup vs baseline: 2.3377x; 1.8190x over previous
import functools

import jax
import jax.numpy as jnp
from jax import lax
from jax.experimental import pallas as pl
from jax.experimental.pallas import tpu as pltpu
from jax.experimental.pallas import tpu_sc as plsc

_ROWS = 16384
_COLS = 200
_NC = 2
_NS = 16
_NW = _NC * _NS
_ROWS_W = _ROWS // _NW
_CROWS = 64
_NCHUNK = _ROWS_W // _CROWS
_LANES = 16
_CSTARTS = tuple(range(0, _COLS - _LANES, _LANES)) + (_COLS - _LANES,)
_GDN = lax.GatherDimensionNumbers(
    offset_dims=(), collapsed_slice_dims=(0,), start_index_map=(0,))

_mesh = plsc.VectorSubcoreMesh(core_axis_name="c", subcore_axis_name="s")


@functools.partial(
    pl.kernel,
    mesh=_mesh,
    out_type=jax.ShapeDtypeStruct((_ROWS, _COLS), jnp.float32),
    compiler_params=pltpu.CompilerParams(use_tc_tiling_on_sc=True),
    scratch_types=[
        pltpu.VMEM((_LANES,), jnp.float32),
        pltpu.VMEM((2, _CROWS, _COLS), jnp.int32),
        pltpu.VMEM((2, _CROWS, _COLS), jnp.float32),
        pltpu.SemaphoreType.DMA,
        pltpu.SemaphoreType.DMA,
        pltpu.SemaphoreType.DMA,
        pltpu.SemaphoreType.DMA,
    ],
)
def _lookup(keys_hbm, vals_hbm, out_hbm, vals_v, in_v, out_v,
            isem0, isem1, osem0, osem1):
    wid = lax.axis_index("s") * _NC + lax.axis_index("c")
    base = wid * _ROWS_W
    isems = (isem0, isem1)
    osems = (osem0, osem1)

    pltpu.sync_copy(vals_hbm, vals_v)
    vals_vec = vals_v[...]

    def in_copy(ci):
        b = ci % 2
        return pltpu.make_async_copy(
            keys_hbm.at[pl.ds(base + ci * _CROWS, _CROWS), :],
            in_v.at[b], isems[b])

    def out_copy(ci):
        b = ci % 2
        return pltpu.make_async_copy(
            out_v.at[b], out_hbm.at[pl.ds(base + ci * _CROWS, _CROWS), :],
            osems[b])

    in_copy(0).start()
    for ci in range(_NCHUNK):
        b = ci % 2
        if ci + 1 < _NCHUNK:
            in_copy(ci + 1).start()
        in_copy(ci).wait()
        if ci >= 2:
            out_copy(ci - 2).wait()
        in_ref = in_v.at[b]
        out_ref = out_v.at[b]

        @plsc.parallel_loop(0, _CROWS, step=1)
        def row_body(r):
            for c in _CSTARTS:
                sl = pl.ds(c, _LANES)
                x = in_ref[r, sl]
                u = plsc.bitcast(x, jnp.uint32)
                idx = plsc.bitcast(jnp.minimum(u, jnp.uint32(3)), jnp.int32)
                out_ref[r, sl] = lax.gather(
                    vals_vec, idx[:, None], dimension_numbers=_GDN,
                    slice_sizes=(1,),
                    mode=lax.GatherScatterMode.PROMISE_IN_BOUNDS)

        out_copy(ci).start()
    out_copy(_NCHUNK - 2).wait()
    out_copy(_NCHUNK - 1).wait()


def kernel(inputs, values):
    vals16 = jnp.zeros((_LANES,), jnp.float32).at[:4].set(values)
    return _lookup(inputs.astype(jnp.int32), vals16)


# speedup vs baseline: 3.4690x; 1.4840x over previous
import functools

import jax
import jax.numpy as jnp
from jax import lax
from jax.experimental import pallas as pl
from jax.experimental.pallas import tpu as pltpu
from jax.experimental.pallas import tpu_sc as plsc

_ROWS = 200
_COLS = 16384
_NC = 2
_NS = 16
_NW = _NC * _NS
_COLS_W = _COLS // _NW
_CROWS = 8
_NCHUNK = _ROWS // _CROWS
_LANES = 16
_GDN = lax.GatherDimensionNumbers(
    offset_dims=(), collapsed_slice_dims=(0,), start_index_map=(0,))

_mesh = plsc.VectorSubcoreMesh(core_axis_name="c", subcore_axis_name="s")


@functools.partial(
    pl.kernel,
    mesh=_mesh,
    out_type=jax.ShapeDtypeStruct((_ROWS, _COLS), jnp.float32),
    compiler_params=pltpu.CompilerParams(use_tc_tiling_on_sc=True),
    scratch_types=[
        pltpu.VMEM((_LANES,), jnp.float32),
        pltpu.VMEM((2, _CROWS, _COLS_W), jnp.int32),
        pltpu.VMEM((2, _CROWS, _COLS_W), jnp.float32),
        pltpu.SemaphoreType.DMA,
        pltpu.SemaphoreType.DMA,
        pltpu.SemaphoreType.DMA,
        pltpu.SemaphoreType.DMA,
    ],
)
def _lookup(keys_hbm, vals_hbm, out_hbm, vals_v, in_v, out_v,
            isem0, isem1, osem0, osem1):
    wid = lax.axis_index("s") * _NC + lax.axis_index("c")
    col0 = wid * _COLS_W
    isems = (isem0, isem1)
    osems = (osem0, osem1)

    pltpu.sync_copy(vals_hbm, vals_v)
    vals_vec = vals_v[...]

    def in_copy(ci):
        b = ci % 2
        return pltpu.make_async_copy(
            keys_hbm.at[pl.ds(ci * _CROWS, _CROWS), pl.ds(col0, _COLS_W)],
            in_v.at[b], isems[b])

    def out_copy(ci):
        b = ci % 2
        return pltpu.make_async_copy(
            out_v.at[b],
            out_hbm.at[pl.ds(ci * _CROWS, _CROWS), pl.ds(col0, _COLS_W)],
            osems[b])

    in_copy(0).start()
    for ci in range(_NCHUNK):
        b = ci % 2
        if ci + 1 < _NCHUNK:
            in_copy(ci + 1).start()
        in_copy(ci).wait()
        if ci >= 2:
            out_copy(ci - 2).wait()
        in_ref = in_v.at[b]
        out_ref = out_v.at[b]

        @plsc.parallel_loop(0, _COLS_W, step=_LANES)
        def col_body(c):
            sl = pl.ds(c, _LANES)
            for r in range(_CROWS):
                x = in_ref[r, sl]
                u = plsc.bitcast(x, jnp.uint32)
                idx = plsc.bitcast(jnp.minimum(u, jnp.uint32(3)), jnp.int32)
                out_ref[r, sl] = lax.gather(
                    vals_vec, idx[:, None], dimension_numbers=_GDN,
                    slice_sizes=(1,),
                    mode=lax.GatherScatterMode.PROMISE_IN_BOUNDS)

        out_copy(ci).start()
    out_copy(_NCHUNK - 2).wait()
    out_copy(_NCHUNK - 1).wait()


def kernel(inputs, values):
    vals16 = jnp.zeros((_LANES,), jnp.float32).at[:4].set(values)
    out_t = _lookup(inputs.astype(jnp.int32).T, vals16)
    return out_t.T
